# baseline (speedup 1.0000x reference)
import functools

import jax
import jax.numpy as jnp
from jax import lax
from jax.experimental import pallas as pl
from jax.experimental.pallas import tpu as pltpu
from jax.experimental.pallas import tpu_sc as plsc

B, T, N, FIN, FOUT, E = 2, 12, 10000, 128, 128, 320000
NBT = B * T
NC, NS = 2, 16
SLICES_PER_CORE = NBT // NC
NP = 10240
ROWS_PER_TILE = NP // NS
G = 64
NB = 320
EP_TILE = NB * G
EPAD = EP_TILE * NS
LANES = 16
NE = 5
NGB = 4
STEPS = 20
ROUNDS = NB // STEPS
CHUNKS = [(i * G, G) for i in range(ROWS_PER_TILE // G)]

def _mm_body(x_ref, wt_ref, b_ref, o_ref):
    o_ref[...] = (
        jnp.dot(x_ref[...], wt_ref[...], preferred_element_type=jnp.float32)
        + b_ref[...]
    )


def _linear(x_flat, Wt, b_row):
    rows = x_flat.shape[0]
    blk = 1000
    grid = rows // blk
    return pl.pallas_call(
        _mm_body,
        grid=(grid,),
        in_specs=[
            pl.BlockSpec((blk, FIN), lambda i: (i, 0)),
            pl.BlockSpec((FIN, FOUT), lambda i: (0, 0)),
            pl.BlockSpec((1, FOUT), lambda i: (0, 0)),
        ],
        out_specs=pl.BlockSpec((blk, FOUT), lambda i: (i, 0)),
        out_shape=jax.ShapeDtypeStruct((rows, FOUT), jnp.float32),
    )(x_flat, Wt, b_row)


def _sc_body(h_hbm, rows_hbm, cols_hbm, vals_hbm, out_hbm, *scr):
    ridx = scr[0:NE]
    gidx = scr[NE:2 * NE]
    vbuf = scr[2 * NE:3 * NE]
    gbuf = scr[3 * NE:3 * NE + NGB]
    esem = scr[3 * NE + NGB:4 * NE + NGB]
    gsem = scr[4 * NE + NGB:4 * NE + 2 * NGB]
    ssem = scr[4 * NE + 2 * NGB:4 * NE + 3 * NGB]
    accum = scr[4 * NE + 3 * NGB]

    cid = lax.axis_index("c")
    sid = lax.axis_index("s")
    row0 = sid * ROWS_PER_TILE
    ebase = sid * EP_TILE

    def _issue_eload(e, b):
        eb = ebase + b * G
        pltpu.async_copy(rows_hbm.at[pl.ds(eb, G)], ridx[e], esem[e])
        pltpu.async_copy(cols_hbm.at[pl.ds(eb, G)], gidx[e], esem[e])
        pltpu.async_copy(vals_hbm.at[pl.ds(eb, G)], vbuf[e], esem[e])

    def _wait_eload(e):
        pltpu.make_async_copy(rows_hbm.at[pl.ds(0, G)], ridx[e], esem[e]).wait()
        pltpu.make_async_copy(cols_hbm.at[pl.ds(0, G)], gidx[e], esem[e]).wait()
        pltpu.make_async_copy(vals_hbm.at[pl.ds(0, G)], vbuf[e], esem[e]).wait()

    H2 = G // 2

    def _issue_gather(e, a):
        pltpu.async_copy(h_hbm.at[gidx[e].at[pl.ds(0, H2)]],
                         gbuf[a].at[pl.ds(0, H2)], gsem[a])
        pltpu.async_copy(h_hbm.at[gidx[e].at[pl.ds(H2, H2)]],
                         gbuf[a].at[pl.ds(H2, H2)], gsem[a])

    def _wait_gather(a, e):
        pltpu.make_async_copy(h_hbm.at[gidx[e].at[pl.ds(0, H2)]],
                              gbuf[a].at[pl.ds(0, H2)], gsem[a]).wait()
        pltpu.make_async_copy(h_hbm.at[gidx[e].at[pl.ds(H2, H2)]],
                              gbuf[a].at[pl.ds(H2, H2)], gsem[a]).wait()

    def _wait_scatter(a, e):
        pltpu.make_async_copy(gbuf[a], accum.at[ridx[e]], ssem[a]).wait()

    def _add_hbase(e, hbase):
        def _gi(kk, _):
            sl = pl.ds(kk * LANES, LANES)
            gidx[e][sl] = gidx[e][sl] + hbase
            return _
        lax.fori_loop(0, G // LANES, _gi, None)

    def _scale(a, e):
        def _scale16(gg, _):
            g0 = gg * LANES
            v16 = vbuf[e][pl.ds(g0, LANES)]
            for g2 in range(LANES):
                v = v16[g2]
                for kk in range(FOUT // LANES):
                    sl = pl.ds(kk * LANES, LANES)
                    gbuf[a][g0 + g2, sl] = gbuf[a][g0 + g2, sl] * v
            return _
        lax.fori_loop(0, G // LANES, _scale16, None)

    def _zero_gbuf0(rows):
        def _zrow(r, _):
            for kk in range(FOUT // LANES):
                gbuf[0][r, pl.ds(kk * LANES, LANES)] = (
                    jnp.zeros((LANES,), jnp.float32))
            return _
        lax.fori_loop(0, rows, _zrow, None)

    def _zero_accum_rows():
        for (r0, ch) in CHUNKS:
            pltpu.async_copy(gbuf[0].at[pl.ds(0, ch)],
                             accum.at[pl.ds(row0 + r0, ch)], ssem[0])
        for (r0, ch) in CHUNKS:
            pltpu.make_async_copy(gbuf[0].at[pl.ds(0, ch)],
                                  accum.at[pl.ds(row0 + r0, ch)],
                                  ssem[0]).wait()

    _zero_gbuf0(G)
    _zero_accum_rows()
    plsc.subcore_barrier()

    def _slice_step(s, _):
        c = cid * SLICES_PER_CORE + s
        hbase = c * N
        obase = c * NP

        for b0 in range(3):
            _issue_eload(b0, b0)
        for b0 in range(2):
            _wait_eload(b0)
            _add_hbase(b0, hbase)
            _issue_gather(b0, b0)

        def _half(r, jj):
            k = r * STEPS + jj
            a = jj % NGB
            e = jj % NE
            a2 = (jj + 2) % NGB
            e2 = (jj + 2) % NE
            ap = (jj - 2) % NGB
            ep = (jj - 2) % NE

            _wait_gather(a, e)
            _scale(a, e)
            pltpu.async_copy(gbuf[a], accum.at[ridx[e]], ssem[a], add=True)
            @pl.when(k >= 2)
            def _():
                _wait_scatter(ap, ep)
            @pl.when(k <= NB - 3)
            def _():
                _wait_eload(e2)
                _add_hbase(e2, hbase)
                _issue_gather(e2, a2)
            @pl.when(k <= NB - 4)
            def _():
                _issue_eload((jj + 3) % NE, k + 3)

        def _round(r, _):
            for jj in range(STEPS):
                _half(r, jj)
            return _
        lax.fori_loop(0, ROUNDS, _round, None)
        _wait_scatter((NB - 2) % NGB, (NB - 2) % NE)
        _wait_scatter((NB - 1) % NGB, (NB - 1) % NE)
        plsc.subcore_barrier()

        for (r0, ch) in CHUNKS:
            pltpu.sync_copy(accum.at[pl.ds(row0 + r0, ch)],
                            gbuf[0].at[pl.ds(0, ch)])

            def _relu(r, _):
                for kk in range(FOUT // LANES):
                    sl = pl.ds(kk * LANES, LANES)
                    gbuf[0][r, sl] = jnp.maximum(gbuf[0][r, sl], 0.0)
                return _
            lax.fori_loop(0, ch, _relu, None)
            pltpu.sync_copy(gbuf[0].at[pl.ds(0, ch)],
                            out_hbm.at[pl.ds(obase + row0 + r0, ch)])

        _zero_gbuf0(G)
        _zero_accum_rows()
        plsc.subcore_barrier()
        return _

    lax.fori_loop(0, SLICES_PER_CORE, _slice_step, None)


_sc_aggregate = functools.partial(
    pl.kernel,
    out_type=jax.ShapeDtypeStruct((NBT * NP, FOUT), jnp.float32),
    mesh=plsc.VectorSubcoreMesh(core_axis_name="c", subcore_axis_name="s"),
    scratch_types=(
        [pltpu.VMEM((G,), jnp.int32) for _ in range(NE)]
        + [pltpu.VMEM((G,), jnp.int32) for _ in range(NE)]
        + [pltpu.VMEM((G,), jnp.float32) for _ in range(NE)]
        + [pltpu.VMEM((G, FOUT), jnp.float32) for _ in range(NGB)]
        + [pltpu.SemaphoreType.DMA for _ in range(NE + 2 * NGB)]
        + [pltpu.VMEM_SHARED((NP, FOUT), jnp.float32)]
    ),
)(_sc_body)


def kernel(x, adj_rows, adj_cols, adj_vals, W, b):
    x_flat = x.reshape(B * T * N, FIN)
    h = _linear(x_flat, W.T, b.reshape(1, FOUT))

    pad = EPAD - E
    rows_p = jnp.pad(adj_rows, (0, pad))
    cols_p = jnp.pad(adj_cols, (0, pad))
    vals_p = jnp.pad(adj_vals, (0, pad))

    out_flat = _sc_aggregate(h, rows_p, cols_p, vals_p)
    return out_flat.reshape(NBT, NP, FOUT)[:, :N].reshape(B, T, N, FOUT)


# speedup vs baseline: 1.0084x; 1.0084x over previous
import functools

import jax
import jax.numpy as jnp
from jax import lax
from jax.experimental import pallas as pl
from jax.experimental.pallas import tpu as pltpu
from jax.experimental.pallas import tpu_sc as plsc

B, T, N, FIN, FOUT, E = 2, 12, 10000, 128, 128, 320000
NBT = B * T
NC, NS = 2, 16
SLICES_PER_CORE = NBT // NC
NP = 10240
ROWS_PER_TILE = NP // NS
G = 64
NB = 320
EP_TILE = NB * G
EPAD = EP_TILE * NS
LANES = 16
NE = 5
NGB = 4
STEPS = 20
ROUNDS = NB // STEPS
CHUNKS = [(i * G, G) for i in range(ROWS_PER_TILE // G)]

def _mm_body(x_ref, wt_ref, b_ref, o_ref):
    o_ref[...] = (
        jnp.dot(x_ref[...], wt_ref[...], preferred_element_type=jnp.float32)
        + b_ref[...]
    )


def _linear(x_flat, Wt, b_row):
    rows = x_flat.shape[0]
    blk = 1000
    grid = rows // blk
    return pl.pallas_call(
        _mm_body,
        grid=(grid,),
        in_specs=[
            pl.BlockSpec((blk, FIN), lambda i: (i, 0)),
            pl.BlockSpec((FIN, FOUT), lambda i: (0, 0)),
            pl.BlockSpec((1, FOUT), lambda i: (0, 0)),
        ],
        out_specs=pl.BlockSpec((blk, FOUT), lambda i: (i, 0)),
        out_shape=jax.ShapeDtypeStruct((rows, FOUT), jnp.float32),
    )(x_flat, Wt, b_row)


def _sc_body(h_hbm, rows_hbm, cols_hbm, vals_hbm, out_hbm, *scr):
    ridx = scr[0:NE]
    gidx = scr[NE:2 * NE]
    vbuf = scr[2 * NE:3 * NE]
    gbuf = scr[3 * NE:3 * NE + NGB]
    esem = scr[3 * NE + NGB:4 * NE + NGB]
    gsem = scr[4 * NE + NGB:4 * NE + 2 * NGB]
    ssem = scr[4 * NE + 2 * NGB:4 * NE + 3 * NGB]
    accum = scr[4 * NE + 3 * NGB]

    cid = lax.axis_index("c")
    sid = lax.axis_index("s")
    row0 = sid * ROWS_PER_TILE
    ebase = sid * EP_TILE

    def _issue_eload(e, b):
        eb = ebase + b * G
        pltpu.async_copy(rows_hbm.at[pl.ds(eb, G)], ridx[e], esem[e])
        pltpu.async_copy(cols_hbm.at[pl.ds(eb, G)], gidx[e], esem[e])
        pltpu.async_copy(vals_hbm.at[pl.ds(eb, G)], vbuf[e], esem[e])

    def _wait_eload(e):
        pltpu.make_async_copy(rows_hbm.at[pl.ds(0, G)], ridx[e], esem[e]).wait()
        pltpu.make_async_copy(cols_hbm.at[pl.ds(0, G)], gidx[e], esem[e]).wait()
        pltpu.make_async_copy(vals_hbm.at[pl.ds(0, G)], vbuf[e], esem[e]).wait()

    def _issue_gather(e, a):
        pltpu.async_copy(h_hbm.at[gidx[e]], gbuf[a], gsem[a])

    def _wait_gather(a, e):
        pltpu.make_async_copy(h_hbm.at[gidx[e]], gbuf[a], gsem[a]).wait()

    def _wait_scatter(a, e):
        pltpu.make_async_copy(gbuf[a], accum.at[ridx[e]], ssem[a]).wait()

    def _add_hbase(e, hbase):
        def _gi(kk, _):
            sl = pl.ds(kk * LANES, LANES)
            gidx[e][sl] = gidx[e][sl] + hbase
            return _
        lax.fori_loop(0, G // LANES, _gi, None)

    def _scale(a, e):
        def _scale16(gg, _):
            g0 = gg * LANES
            v16 = vbuf[e][pl.ds(g0, LANES)]
            for g2 in range(LANES):
                v = v16[g2]
                for kk in range(FOUT // LANES):
                    sl = pl.ds(kk * LANES, LANES)
                    gbuf[a][g0 + g2, sl] = gbuf[a][g0 + g2, sl] * v
            return _
        lax.fori_loop(0, G // LANES, _scale16, None)

    def _zero_gbuf0(rows):
        def _zrow(r, _):
            for kk in range(FOUT // LANES):
                gbuf[0][r, pl.ds(kk * LANES, LANES)] = (
                    jnp.zeros((LANES,), jnp.float32))
            return _
        lax.fori_loop(0, rows, _zrow, None)

    def _zero_accum_rows():
        for (r0, ch) in CHUNKS:
            pltpu.async_copy(gbuf[0].at[pl.ds(0, ch)],
                             accum.at[pl.ds(row0 + r0, ch)], ssem[0])
        for (r0, ch) in CHUNKS:
            pltpu.make_async_copy(gbuf[0].at[pl.ds(0, ch)],
                                  accum.at[pl.ds(row0 + r0, ch)],
                                  ssem[0]).wait()

    _zero_gbuf0(G)
    _zero_accum_rows()
    plsc.subcore_barrier()

    def _slice_step(s, _):
        c = cid * SLICES_PER_CORE + s
        hbase = c * N
        obase = c * NP

        for b0 in range(3):
            _issue_eload(b0, b0)
        for b0 in range(2):
            _wait_eload(b0)
            _add_hbase(b0, hbase)
            _issue_gather(b0, b0)

        def _half(r, jj):
            k = r * STEPS + jj
            a = jj % NGB
            e = jj % NE
            a2 = (jj + 2) % NGB
            e2 = (jj + 2) % NE
            ap = (jj - 2) % NGB
            ep = (jj - 2) % NE

            _wait_gather(a, e)
            _scale(a, e)
            pltpu.async_copy(gbuf[a], accum.at[ridx[e]], ssem[a], add=True)
            @pl.when(k >= 2)
            def _():
                _wait_scatter(ap, ep)
            @pl.when(k <= NB - 3)
            def _():
                _wait_eload(e2)
                _add_hbase(e2, hbase)
                _issue_gather(e2, a2)
            @pl.when(k <= NB - 4)
            def _():
                _issue_eload((jj + 3) % NE, k + 3)

        def _round(r, _):
            for jj in range(STEPS):
                _half(r, jj)
            return _
        lax.fori_loop(0, ROUNDS, _round, None)
        _wait_scatter((NB - 2) % NGB, (NB - 2) % NE)
        _wait_scatter((NB - 1) % NGB, (NB - 1) % NE)
        plsc.subcore_barrier()

        for (r0, ch) in CHUNKS:
            pltpu.sync_copy(accum.at[pl.ds(row0 + r0, ch)],
                            gbuf[0].at[pl.ds(0, ch)])

            def _relu(r, _):
                for kk in range(FOUT // LANES):
                    sl = pl.ds(kk * LANES, LANES)
                    gbuf[0][r, sl] = jnp.maximum(gbuf[0][r, sl], 0.0)
                return _
            lax.fori_loop(0, ch, _relu, None)
            pltpu.sync_copy(gbuf[0].at[pl.ds(0, ch)],
                            out_hbm.at[pl.ds(obase + row0 + r0, ch)])

        _zero_gbuf0(G)
        _zero_accum_rows()
        plsc.subcore_barrier()
        return _

    lax.fori_loop(0, SLICES_PER_CORE, _slice_step, None)


_sc_aggregate = functools.partial(
    pl.kernel,
    out_type=jax.ShapeDtypeStruct((NBT * NP, FOUT), jnp.float32),
    mesh=plsc.VectorSubcoreMesh(core_axis_name="c", subcore_axis_name="s"),
    scratch_types=(
        [pltpu.VMEM((G,), jnp.int32) for _ in range(NE)]
        + [pltpu.VMEM((G,), jnp.int32) for _ in range(NE)]
        + [pltpu.VMEM((G,), jnp.float32) for _ in range(NE)]
        + [pltpu.VMEM((G, FOUT), jnp.float32) for _ in range(NGB)]
        + [pltpu.SemaphoreType.DMA for _ in range(NE + 2 * NGB)]
        + [pltpu.VMEM_SHARED((NP, FOUT), jnp.float32)]
    ),
)(_sc_body)


def kernel(x, adj_rows, adj_cols, adj_vals, W, b):
    x_flat = x.reshape(B * T * N, FIN)
    h = _linear(x_flat, W.T, b.reshape(1, FOUT))

    pad = EPAD - E
    rows_p = jnp.pad(adj_rows, (0, pad))
    cols_p = jnp.pad(adj_cols, (0, pad))
    vals_p = jnp.pad(adj_vals, (0, pad))

    out_flat = _sc_aggregate(h, rows_p, cols_p, vals_p)
    return out_flat.reshape(NBT, NP, FOUT)[:, :N].reshape(B, T, N, FOUT)
